# Initial kernel scaffold; baseline (speedup 1.0000x reference)
import jax
import jax.numpy as jnp
from jax.experimental import pallas as pl


def kernel(x, edge_index, bn_w, bn_b, in1_W, in1_b, in2_W, in2_b, conv1_W, conv1_b, conv2_W, conv2_b, edge1_W, edge1_b, edge2_W, edge2_b):
    raise NotImplementedError("write your pallas kernel here")


# speedup vs baseline: 2.2268x; 2.2268x over previous
import functools

import jax
import jax.numpy as jnp
from jax import lax
from jax.experimental import pallas as pl
from jax.experimental.pallas import tpu as pltpu
from jax.experimental.pallas import tpu_sc as plsc

N = 10000
E = 320000
D = 128
HD = 128
F = HD + D
_NC = 2
_NS = 16
_NW = _NC * _NS
_EPW = E // _NW
_BG = 80
_NBG = _EPW // _BG
_NPAD = 10240
_RPS = _NPAD // _NS

_f32 = jnp.float32


def _mesh():
    return plsc.VectorSubcoreMesh(
        core_axis_name="c", subcore_axis_name="s",
        num_cores=_NC, num_subcores=_NS)


def _bn_body(x_ref, w_ref, b_ref, o_ref):
    x = x_ref[...]
    mu = jnp.mean(x, axis=0, keepdims=True)
    xc = x - mu
    var = jnp.mean(xc * xc, axis=0, keepdims=True)
    o_ref[...] = xc * lax.rsqrt(var + 1e-5) * w_ref[...] + b_ref[...]


def _node1_body(xb_ref, w1_ref, b1_ref, w2_ref, b2_ref, cw_ref, cb_ref,
                p_ref, q_ref):
    xb = xb_ref[...]
    t = jnp.maximum(
        jnp.dot(xb, w1_ref[...], preferred_element_type=_f32) + b1_ref[...],
        0.0)
    h0 = jnp.tanh(
        jnp.dot(t, w2_ref[...], preferred_element_type=_f32) + b2_ref[...])
    feat = jnp.concatenate([h0, xb], axis=1)
    cw = cw_ref[...]
    wb = cw[F:]
    wa = cw[:F] - wb
    p_ref[...] = jnp.dot(feat, wa, preferred_element_type=_f32) + cb_ref[...]
    q_ref[...] = jnp.dot(feat, wb, preferred_element_type=_f32)


def _conv2_body(g_ref, w_ref, b_ref, m_ref):
    m_ref[...] = jnp.tanh(
        jnp.dot(g_ref[...], w_ref[...], preferred_element_type=_f32)
        + b_ref[...])


def _edgered_body(u_ref, b2_ref, o_ref):
    ones = jnp.ones((1, 16), _f32)
    z = lax.dot_general(ones, u_ref[...], (((1,), (1,)), ((), ())),
                        preferred_element_type=_f32)
    o_ref[...] = (1.0 / (1.0 + jnp.exp(-(z + b2_ref[0, 0])))).reshape(
        1, 1, z.shape[1])


def _node2_body(h1_ref, h2_ref, xb_ref, w_ref, b_ref, r_ref, s_ref):
    feat = jnp.concatenate([h1_ref[0] + h2_ref[0], xb_ref[...]], axis=1)
    w = w_ref[...]
    r_ref[...] = jnp.dot(feat, w[:F], preferred_element_type=_f32) + b_ref[...]
    s_ref[...] = jnp.dot(feat, w[F:], preferred_element_type=_f32)


def _sc_gather_relu(p_hbm, qq_hbm, ia_hbm, ib_hbm, g_hbm,
                    iav, ibv, bufp, bufq, semp, semq):
    wid = lax.axis_index("s") * _NC + lax.axis_index("c")
    base = wid * _EPW

    @pl.loop(0, _NBG)
    def _blk(i):
        off = base + i * _BG
        pltpu.sync_copy(ia_hbm.at[pl.ds(off, _BG)], iav)
        pltpu.sync_copy(ib_hbm.at[pl.ds(off, _BG)], ibv)
        cp1 = pltpu.async_copy(p_hbm.at[iav], bufp, semp)
        cp2 = pltpu.async_copy(qq_hbm.at[ibv], bufq, semq)
        cp1.wait()
        cp2.wait()

        @pl.loop(0, _BG)
        def _row(r):
            for ch in range(F // 16):
                sl = pl.ds(ch * 16, 16)
                bufp[r, sl] = jnp.maximum(bufp[r, sl] + bufq[r, sl], 0.0)

        pltpu.sync_copy(bufp, g_hbm.at[pl.ds(off, _BG)])


def _sc_scatter_add(msg_hbm, col_hbm, hp_hbm, colv, mbuf, zbuf, acc):
    cidx = lax.axis_index("c")
    sidx = lax.axis_index("s")
    wid = sidx * _NC + cidx

    @pl.loop(0, 128)
    def _z(r):
        for ch in range(D // 16):
            zbuf[r, pl.ds(ch * 16, 16)] = jnp.zeros((16,), _f32)

    for j in range(_RPS // 128):
        pltpu.sync_copy(zbuf, acc.at[pl.ds(sidx * _RPS + j * 128, 128)])
    plsc.subcore_barrier()

    base = wid * _EPW

    @pl.loop(0, _NBG)
    def _blk(i):
        off = base + i * _BG
        pltpu.sync_copy(col_hbm.at[pl.ds(off, _BG)], colv)
        pltpu.sync_copy(msg_hbm.at[pl.ds(off, _BG)], mbuf)
        pltpu.sync_copy(mbuf, acc.at[colv], add=True)

    plsc.subcore_barrier()
    for j in range(_RPS // 128):
        r0 = sidx * _RPS + j * 128
        pltpu.sync_copy(acc.at[pl.ds(r0, 128)],
                        hp_hbm.at[cidx, pl.ds(r0, 128)])


def _sc_edge_partial(r_hbm, s_hbm, ia_hbm, ib_hbm, w2_hbm, up_hbm,
                     iav, ibv, bufr, bufs, w2v, outv, semr, sems):
    wid = lax.axis_index("s") * _NC + lax.axis_index("c")
    pltpu.sync_copy(w2_hbm, w2v)
    base = wid * _EPW

    @pl.loop(0, _NBG)
    def _blk(i):
        off = base + i * _BG
        pltpu.sync_copy(ia_hbm.at[pl.ds(off, _BG)], iav)
        pltpu.sync_copy(ib_hbm.at[pl.ds(off, _BG)], ibv)
        cp1 = pltpu.async_copy(r_hbm.at[iav], bufr, semr)
        cp2 = pltpu.async_copy(s_hbm.at[ibv], bufs, sems)
        cp1.wait()
        cp2.wait()

        @pl.loop(0, _BG)
        def _edge(e):
            acc = jnp.zeros((16,), _f32)
            for ch in range(F // 16):
                sl = pl.ds(ch * 16, 16)
                t = jnp.maximum(bufr[e, sl] + bufs[e, sl], 0.0)
                acc = acc + t * w2v[sl]
            outv[e, :] = acc

        pltpu.sync_copy(outv, up_hbm.at[pl.ds(off, _BG)])


@jax.jit
def kernel(x, edge_index, bn_w, bn_b, in1_W, in1_b, in2_W, in2_b,
           conv1_W, conv1_b, conv2_W, conv2_b, edge1_W, edge1_b,
           edge2_W, edge2_b):
    row = edge_index[0]
    col = edge_index[1]

    X = pl.pallas_call(
        _bn_body,
        out_shape=jax.ShapeDtypeStruct((N, D), _f32),
    )(x, bn_w.reshape(1, D), bn_b.reshape(1, D))

    nblk = 5
    rb = N // nblk
    full = lambda shape: pl.BlockSpec(shape, lambda i: (0, 0))
    p, q = pl.pallas_call(
        _node1_body,
        grid=(nblk,),
        in_specs=[
            pl.BlockSpec((rb, D), lambda i: (i, 0)),
            full((D, HD)), full((1, HD)), full((HD, HD)), full((1, HD)),
            full((2 * F, F)), full((1, F)),
        ],
        out_specs=[pl.BlockSpec((rb, F), lambda i: (i, 0))] * 2,
        out_shape=[jax.ShapeDtypeStruct((N, F), _f32)] * 2,
    )(X, in1_W, in1_b.reshape(1, HD), in2_W, in2_b.reshape(1, HD),
      conv1_W, conv1_b.reshape(1, F))

    g = pl.kernel(
        _sc_gather_relu,
        out_type=jax.ShapeDtypeStruct((E, F), _f32),
        mesh=_mesh(),
        scratch_types=[
            pltpu.VMEM((_BG,), jnp.int32),
            pltpu.VMEM((_BG,), jnp.int32),
            pltpu.VMEM((_BG, F), _f32),
            pltpu.VMEM((_BG, F), _f32),
            pltpu.SemaphoreType.DMA,
            pltpu.SemaphoreType.DMA,
        ],
    )(p, q, col, row)

    eblk = 2560
    msg = pl.pallas_call(
        _conv2_body,
        grid=(E // eblk,),
        in_specs=[
            pl.BlockSpec((eblk, F), lambda i: (i, 0)),
            full((F, HD)), full((1, HD)),
        ],
        out_specs=pl.BlockSpec((eblk, HD), lambda i: (i, 0)),
        out_shape=jax.ShapeDtypeStruct((E, HD), _f32),
    )(g, conv2_W, conv2_b.reshape(1, HD))

    hp = pl.kernel(
        _sc_scatter_add,
        out_type=jax.ShapeDtypeStruct((_NC, _NPAD, D), _f32),
        mesh=_mesh(),
        scratch_types=[
            pltpu.VMEM((_BG,), jnp.int32),
            pltpu.VMEM((_BG, D), _f32),
            pltpu.VMEM((128, D), _f32),
            pltpu.VMEM_SHARED((_NPAD, D), _f32),
        ],
    )(msg, col)

    r, s = pl.pallas_call(
        _node2_body,
        grid=(nblk,),
        in_specs=[
            pl.BlockSpec((1, rb, D), lambda i: (0, i, 0)),
            pl.BlockSpec((1, rb, D), lambda i: (1, i, 0)),
            pl.BlockSpec((rb, D), lambda i: (i, 0)),
            full((2 * F, F)), full((1, F)),
        ],
        out_specs=[pl.BlockSpec((rb, F), lambda i: (i, 0))] * 2,
        out_shape=[jax.ShapeDtypeStruct((N, F), _f32)] * 2,
    )(hp, hp, X, edge1_W, edge1_b.reshape(1, F))

    up = pl.kernel(
        _sc_edge_partial,
        out_type=jax.ShapeDtypeStruct((E, 16), _f32),
        mesh=_mesh(),
        scratch_types=[
            pltpu.VMEM((_BG,), jnp.int32),
            pltpu.VMEM((_BG,), jnp.int32),
            pltpu.VMEM((_BG, F), _f32),
            pltpu.VMEM((_BG, F), _f32),
            pltpu.VMEM((F,), _f32),
            pltpu.VMEM((_BG, 16), _f32),
            pltpu.SemaphoreType.DMA,
            pltpu.SemaphoreType.DMA,
        ],
    )(r, s, row, col, edge2_W.reshape(F))

    dblk = 2560
    out3d = pl.pallas_call(
        _edgered_body,
        grid=(E // dblk,),
        in_specs=[
            pl.BlockSpec((dblk, 16), lambda i: (i, 0)),
            full((1, 1)),
        ],
        out_specs=pl.BlockSpec((1, 1, dblk), lambda i: (i, 0, 0)),
        out_shape=jax.ShapeDtypeStruct((E // dblk, 1, dblk), _f32),
    )(up, edge2_b.reshape(1, 1))

    return out3d.reshape(E)


# speedup vs baseline: 3.8856x; 1.7449x over previous
import functools

import jax
import jax.numpy as jnp
from jax import lax
from jax.experimental import pallas as pl
from jax.experimental.pallas import tpu as pltpu
from jax.experimental.pallas import tpu_sc as plsc

N = 10000
E = 320000
D = 128
HD = 128
F = HD + D
_NC = 2
_NS = 16
_NW = _NC * _NS
_EPW = E // _NW
_BG = 40
_NBG = _EPW // _BG
_BS = 80
_NBS = _EPW // _BS
_NPAD = 10240
_RPS = _NPAD // _NS

_f32 = jnp.float32


def _mesh():
    return plsc.VectorSubcoreMesh(
        core_axis_name="c", subcore_axis_name="s",
        num_cores=_NC, num_subcores=_NS)


def _bn_body(x_ref, w_ref, b_ref, o_ref):
    x = x_ref[...]
    mu = jnp.mean(x, axis=0, keepdims=True)
    xc = x - mu
    var = jnp.mean(xc * xc, axis=0, keepdims=True)
    o_ref[...] = xc * lax.rsqrt(var + 1e-5) * w_ref[...] + b_ref[...]


def _node1_body(xb_ref, w1_ref, b1_ref, w2_ref, b2_ref, cw_ref, cb_ref,
                p_ref, q_ref):
    xb = xb_ref[...]
    t = jnp.maximum(
        jnp.dot(xb, w1_ref[...], preferred_element_type=_f32) + b1_ref[...],
        0.0)
    h0 = jnp.tanh(
        jnp.dot(t, w2_ref[...], preferred_element_type=_f32) + b2_ref[...])
    feat = jnp.concatenate([h0, xb], axis=1)
    cw = cw_ref[...]
    wb = cw[F:]
    wa = cw[:F] - wb
    p_ref[...] = jnp.dot(feat, wa, preferred_element_type=_f32) + cb_ref[...]
    q_ref[...] = jnp.dot(feat, wb, preferred_element_type=_f32)


def _conv2_body(g_ref, w_ref, b_ref, m_ref):
    m_ref[...] = jnp.tanh(
        jnp.dot(g_ref[...], w_ref[...], preferred_element_type=_f32)
        + b_ref[...])


def _edgered_body(u_ref, b2_ref, o_ref):
    ones = jnp.ones((1, 16), _f32)
    z = lax.dot_general(ones, u_ref[...], (((1,), (1,)), ((), ())),
                        preferred_element_type=_f32)
    o_ref[...] = (1.0 / (1.0 + jnp.exp(-(z + b2_ref[0, 0])))).reshape(
        1, 1, z.shape[1])


def _node2_body(h1_ref, h2_ref, xb_ref, w_ref, b_ref, r_ref, s_ref):
    feat = jnp.concatenate([h1_ref[0] + h2_ref[0], xb_ref[...]], axis=1)
    w = w_ref[...]
    r_ref[...] = jnp.dot(feat, w[:F], preferred_element_type=_f32) + b_ref[...]
    s_ref[...] = jnp.dot(feat, w[F:], preferred_element_type=_f32)


def _sc_gather_relu(p_hbm, qq_hbm, ia_hbm, ib_hbm, g_hbm,
                    iaall, iball, bp0, bp1, bq0, bq1, go0, go1,
                    sp0, sp1, sq0, sq1, st0, st1):
    wid = lax.axis_index("s") * _NC + lax.axis_index("c")
    base = wid * _EPW
    pltpu.sync_copy(ia_hbm.at[pl.ds(base, _EPW)], iaall)
    pltpu.sync_copy(ib_hbm.at[pl.ds(base, _EPW)], iball)
    bp, bq, go = (bp0, bp1), (bq0, bq1), (go0, go1)
    sp, sq, st = (sp0, sp1), (sq0, sq1), (st0, st1)

    def issue(i, b):
        pltpu.async_copy(p_hbm.at[iaall.at[pl.ds(i * _BG, _BG)]], bp[b], sp[b])
        pltpu.async_copy(qq_hbm.at[iball.at[pl.ds(i * _BG, _BG)]], bq[b], sq[b])

    issue(0, 0)

    @pl.loop(0, _NBG // 2)
    def _j(j):
        for b in range(2):
            i = j * 2 + b
            nb = 1 - b

            @pl.when(i + 1 < _NBG)
            def _():
                issue(i + 1, nb)

            pltpu.make_async_copy(p_hbm.at[pl.ds(0, _BG)], bp[b], sp[b]).wait()
            pltpu.make_async_copy(qq_hbm.at[pl.ds(0, _BG)], bq[b], sq[b]).wait()

            @pl.when(i >= 2)
            def _():
                pltpu.make_async_copy(
                    go[b], g_hbm.at[pl.ds(0, _BG)], st[b]).wait()

            @pl.loop(0, _BG)
            def _row(r):
                for ch in range(F // 16):
                    sl = pl.ds(ch * 16, 16)
                    go[b][r, sl] = jnp.maximum(
                        bp[b][r, sl] + bq[b][r, sl], 0.0)

            pltpu.async_copy(go[b], g_hbm.at[pl.ds(base + i * _BG, _BG)],
                             st[b])

    for b in range(2):
        pltpu.make_async_copy(go[b], g_hbm.at[pl.ds(0, _BG)], st[b]).wait()


def _sc_scatter_add(msg_hbm, col_hbm, hp_hbm, cv0, cv1, mb0, mb1, zbuf, acc,
                    sc0, sc1, sm0, sm1):
    cidx = lax.axis_index("c")
    sidx = lax.axis_index("s")
    wid = sidx * _NC + cidx

    @pl.loop(0, 128)
    def _z(r):
        for ch in range(D // 16):
            zbuf[r, pl.ds(ch * 16, 16)] = jnp.zeros((16,), _f32)

    for j in range(_RPS // 128):
        pltpu.sync_copy(zbuf, acc.at[pl.ds(sidx * _RPS + j * 128, 128)])
    plsc.subcore_barrier()

    base = wid * _EPW
    cv, mb = (cv0, cv1), (mb0, mb1)
    sc_, sm = (sc0, sc1), (sm0, sm1)

    def issue(i, b):
        off = base + i * _BS
        pltpu.async_copy(col_hbm.at[pl.ds(off, _BS)], cv[b], sc_[b])
        pltpu.async_copy(msg_hbm.at[pl.ds(off, _BS)], mb[b], sm[b])

    def drain(b):
        pltpu.make_async_copy(col_hbm.at[pl.ds(0, _BS)], cv[b], sc_[b]).wait()
        pltpu.make_async_copy(msg_hbm.at[pl.ds(0, _BS)], mb[b], sm[b]).wait()

    issue(0, 0)

    @pl.loop(0, _NBS // 2)
    def _j(j):
        for b in range(2):
            i = j * 2 + b

            @pl.when(i + 1 < _NBS)
            def _():
                issue(i + 1, 1 - b)

            drain(b)
            pltpu.sync_copy(mb[b], acc.at[cv[b]], add=True)

    drain(0)
    pltpu.sync_copy(mb[0], acc.at[cv[0]], add=True)

    plsc.subcore_barrier()
    for j in range(_RPS // 128):
        r0 = sidx * _RPS + j * 128
        pltpu.sync_copy(acc.at[pl.ds(r0, 128)],
                        hp_hbm.at[cidx, pl.ds(r0, 128)])


def _sc_edge_partial(r_hbm, s_hbm, ia_hbm, ib_hbm, w2_hbm, up_hbm,
                     iaall, iball, br0, br1, bs0, bs1, ov0, ov1, w2v,
                     sp0, sp1, sq0, sq1, st0, st1):
    wid = lax.axis_index("s") * _NC + lax.axis_index("c")
    base = wid * _EPW
    pltpu.sync_copy(w2_hbm, w2v)
    pltpu.sync_copy(ia_hbm.at[pl.ds(base, _EPW)], iaall)
    pltpu.sync_copy(ib_hbm.at[pl.ds(base, _EPW)], iball)
    br, bs, ov = (br0, br1), (bs0, bs1), (ov0, ov1)
    sp, sq, st = (sp0, sp1), (sq0, sq1), (st0, st1)

    def issue(i, b):
        pltpu.async_copy(r_hbm.at[iaall.at[pl.ds(i * _BG, _BG)]], br[b], sp[b])
        pltpu.async_copy(s_hbm.at[iball.at[pl.ds(i * _BG, _BG)]], bs[b], sq[b])

    issue(0, 0)

    @pl.loop(0, _NBG // 2)
    def _j(j):
        for b in range(2):
            i = j * 2 + b
            nb = 1 - b

            @pl.when(i + 1 < _NBG)
            def _():
                issue(i + 1, nb)

            pltpu.make_async_copy(r_hbm.at[pl.ds(0, _BG)], br[b], sp[b]).wait()
            pltpu.make_async_copy(s_hbm.at[pl.ds(0, _BG)], bs[b], sq[b]).wait()

            @pl.when(i >= 2)
            def _():
                pltpu.make_async_copy(
                    ov[b], up_hbm.at[pl.ds(0, _BG)], st[b]).wait()

            @pl.loop(0, _BG)
            def _edge(e):
                acc = jnp.zeros((16,), _f32)
                for ch in range(F // 16):
                    sl = pl.ds(ch * 16, 16)
                    t = jnp.maximum(br[b][e, sl] + bs[b][e, sl], 0.0)
                    acc = acc + t * w2v[sl]
                ov[b][e, :] = acc

            pltpu.async_copy(ov[b], up_hbm.at[pl.ds(base + i * _BG, _BG)],
                             st[b])

    for b in range(2):
        pltpu.make_async_copy(ov[b], up_hbm.at[pl.ds(0, _BG)], st[b]).wait()


@jax.jit
def kernel(x, edge_index, bn_w, bn_b, in1_W, in1_b, in2_W, in2_b,
           conv1_W, conv1_b, conv2_W, conv2_b, edge1_W, edge1_b,
           edge2_W, edge2_b):
    row = edge_index[0]
    col = edge_index[1]

    X = pl.pallas_call(
        _bn_body,
        out_shape=jax.ShapeDtypeStruct((N, D), _f32),
    )(x, bn_w.reshape(1, D), bn_b.reshape(1, D))

    nblk = 5
    rb = N // nblk
    full = lambda shape: pl.BlockSpec(shape, lambda i: (0, 0))
    p, q = pl.pallas_call(
        _node1_body,
        grid=(nblk,),
        in_specs=[
            pl.BlockSpec((rb, D), lambda i: (i, 0)),
            full((D, HD)), full((1, HD)), full((HD, HD)), full((1, HD)),
            full((2 * F, F)), full((1, F)),
        ],
        out_specs=[pl.BlockSpec((rb, F), lambda i: (i, 0))] * 2,
        out_shape=[jax.ShapeDtypeStruct((N, F), _f32)] * 2,
    )(X, in1_W, in1_b.reshape(1, HD), in2_W, in2_b.reshape(1, HD),
      conv1_W, conv1_b.reshape(1, F))

    g = pl.kernel(
        _sc_gather_relu,
        out_type=jax.ShapeDtypeStruct((E, F), _f32),
        mesh=_mesh(),
        scratch_types=(
            [pltpu.VMEM((_EPW,), jnp.int32)] * 2
            + [pltpu.VMEM((_BG, F), _f32)] * 6
            + [pltpu.SemaphoreType.DMA] * 6
        ),
    )(p, q, col, row)

    eblk = 2560
    msg = pl.pallas_call(
        _conv2_body,
        grid=(E // eblk,),
        in_specs=[
            pl.BlockSpec((eblk, F), lambda i: (i, 0)),
            full((F, HD)), full((1, HD)),
        ],
        out_specs=pl.BlockSpec((eblk, HD), lambda i: (i, 0)),
        out_shape=jax.ShapeDtypeStruct((E, HD), _f32),
    )(g, conv2_W, conv2_b.reshape(1, HD))

    hp = pl.kernel(
        _sc_scatter_add,
        out_type=jax.ShapeDtypeStruct((_NC, _NPAD, D), _f32),
        mesh=_mesh(),
        scratch_types=(
            [pltpu.VMEM((_BS,), jnp.int32)] * 2
            + [pltpu.VMEM((_BS, D), _f32)] * 2
            + [pltpu.VMEM((128, D), _f32),
               pltpu.VMEM_SHARED((_NPAD, D), _f32)]
            + [pltpu.SemaphoreType.DMA] * 4
        ),
    )(msg, col)

    r, s = pl.pallas_call(
        _node2_body,
        grid=(nblk,),
        in_specs=[
            pl.BlockSpec((1, rb, D), lambda i: (0, i, 0)),
            pl.BlockSpec((1, rb, D), lambda i: (1, i, 0)),
            pl.BlockSpec((rb, D), lambda i: (i, 0)),
            full((2 * F, F)), full((1, F)),
        ],
        out_specs=[pl.BlockSpec((rb, F), lambda i: (i, 0))] * 2,
        out_shape=[jax.ShapeDtypeStruct((N, F), _f32)] * 2,
    )(hp, hp, X, edge1_W, edge1_b.reshape(1, F))

    up = pl.kernel(
        _sc_edge_partial,
        out_type=jax.ShapeDtypeStruct((E, 16), _f32),
        mesh=_mesh(),
        scratch_types=(
            [pltpu.VMEM((_EPW,), jnp.int32)] * 2
            + [pltpu.VMEM((_BG, F), _f32)] * 4
            + [pltpu.VMEM((_BG, 16), _f32)] * 2
            + [pltpu.VMEM((F,), _f32)]
            + [pltpu.SemaphoreType.DMA] * 6
        ),
    )(r, s, row, col, edge2_W.reshape(F))

    dblk = 2560
    out3d = pl.pallas_call(
        _edgered_body,
        grid=(E // dblk,),
        in_specs=[
            pl.BlockSpec((dblk, 16), lambda i: (i, 0)),
            full((1, 1)),
        ],
        out_specs=pl.BlockSpec((1, 1, dblk), lambda i: (i, 0, 0)),
        out_shape=jax.ShapeDtypeStruct((E // dblk, 1, dblk), _f32),
    )(up, edge2_b.reshape(1, 1))

    return out3d.reshape(E)
